# baseline (speedup 1.0000x reference)
import functools

import jax
import jax.numpy as jnp
from jax import lax
from jax.experimental import pallas as pl
from jax.experimental.pallas import tpu as pltpu
from jax.experimental.pallas import tpu_sc as plsc

_VOCAB = 100000
_DIM = 64
_BATCH = 4096
_HIST = 50
_N = _BATCH * _HIST

_NC, _NS = 2, 16
_NW = _NC * _NS
_BBLK = _BATCH // _NW
_NIDX = _BBLK * _HIST
_SLOT = 56
_GRP = 8
_NG = _BBLK // _GRP
_CHUNK_OFFS = (0, 16, 32, 34)


def _body(arr_hbm, table_hbm, oov_hbm, out_hbm,
          raw_v, idx2d, rows0, rows1, oov_v,
          gsem0, gsem1, osem0, osem1):
    wid = lax.axis_index("s") * _NC + lax.axis_index("c")
    base_n = wid * _NIDX
    base_b = wid * _BBLK

    pltpu.sync_copy(arr_hbm.at[pl.ds(base_n, _NIDX)], raw_v)
    pltpu.sync_copy(oov_hbm, oov_v)

    zeros = jnp.zeros((16,), jnp.int32)

    def _sanitize(b, acc):
        idx2d[b, pl.ds(40, 16)] = zeros
        for off in _CHUNK_OFFS:
            v = raw_v[pl.ds(b * _HIST + off, 16)]
            idx2d[b, pl.ds(off, 16)] = jnp.maximum(v, 0)
            acc = jnp.minimum(acc, v)
        return acc
    min_acc = lax.fori_loop(0, _BBLK, _sanitize, jnp.zeros((16,), jnp.int32))
    lane_min = min_acc[0]
    for _j in range(1, 16):
        lane_min = jnp.minimum(lane_min, min_acc[_j])

    bufs = (rows0, rows1)
    gsems = (gsem0, gsem1)
    osems = (osem0, osem1)
    gathers = [None] * _NG
    outcopies = [None] * _NG

    def _drain_and_emit(g):
        buf = bufs[g % 2]
        for h in gathers[g]:
            h.wait()
        outcopies[g] = [
            pltpu.async_copy(
                buf.at[pl.ds(j * _SLOT, _HIST)],
                out_hbm.at[base_b + (g * _GRP + j)],
                osems[g % 2])
            for j in range(_GRP)
        ]

    for g in range(_NG):
        p = g % 2
        if g >= 2:
            for h in outcopies[g - 2]:
                h.wait()
        gathers[g] = [
            pltpu.async_copy(
                table_hbm.at[idx2d.at[g * _GRP + j]],
                bufs[p].at[pl.ds(j * _SLOT, _SLOT)],
                gsems[p])
            for j in range(_GRP)
        ]
        if g >= 1:
            _drain_and_emit(g - 1)
    _drain_and_emit(_NG - 1)
    for g in (_NG - 2, _NG - 1):
        for h in outcopies[g]:
            h.wait()

    @pl.when(lane_min < 0)
    def _():
        def _row(b, carry):
            for off in _CHUNK_OFFS:
                v = raw_v[pl.ds(b * _HIST + off, 16)]
                for j in range(16):
                    if off + j < _HIST:
                        @pl.when(v[j] < 0)
                        def _():
                            pltpu.sync_copy(
                                oov_v, out_hbm.at[base_b + b, off + j])
            return carry
        lax.fori_loop(0, _BBLK, _row, jnp.int32(0))


def kernel(arr, table, oov):
    mesh = plsc.VectorSubcoreMesh(core_axis_name="c", subcore_axis_name="s")
    kern = functools.partial(
        pl.kernel,
        out_type=jax.ShapeDtypeStruct((_BATCH, _HIST, _DIM), jnp.float32),
        mesh=mesh,
        compiler_params=pltpu.CompilerParams(use_tc_tiling_on_sc=False),
        scratch_types=[
            pltpu.VMEM((_NIDX,), jnp.int32),
            pltpu.VMEM((_BBLK, _SLOT), jnp.int32),
            pltpu.VMEM((_GRP * _SLOT, _DIM), jnp.float32),
            pltpu.VMEM((_GRP * _SLOT, _DIM), jnp.float32),
            pltpu.VMEM((_DIM,), jnp.float32),
            pltpu.SemaphoreType.DMA,
            pltpu.SemaphoreType.DMA,
            pltpu.SemaphoreType.DMA,
            pltpu.SemaphoreType.DMA,
        ],
    )(_body)
    return kern(arr.reshape(-1), table, oov)


# speedup vs baseline: 3.1412x; 3.1412x over previous
import functools

import jax
import jax.numpy as jnp
from jax import lax
from jax.experimental import pallas as pl
from jax.experimental.pallas import tpu as pltpu
from jax.experimental.pallas import tpu_sc as plsc

_VOCAB = 100000
_DIM = 64
_BATCH = 4096
_HIST = 50
_N = _BATCH * _HIST

_NC, _NS = 2, 16
_NW = _NC * _NS
_BPW = _N // _NW
_STREAM = 128
_ROWS_PER_GROUP = 256
_NBUF = 4
_SPG = _ROWS_PER_GROUP // _STREAM
_NG = _BPW // _ROWS_PER_GROUP
_IDX_ROWS = _BPW // _STREAM


def _body(arr_hbm, table_hbm, oov_hbm, out_hbm,
          raw_v, idx2d, rows0, rows1, rows2, rows3, oov_v,
          gsem0, gsem1, gsem2, gsem3, osem0, osem1, osem2, osem3):
    wid = lax.axis_index("s") * _NC + lax.axis_index("c")
    base = wid * _BPW

    pltpu.sync_copy(arr_hbm.at[pl.ds(base, _BPW)], raw_v)
    pltpu.sync_copy(oov_hbm, oov_v)

    def _sanitize(r, acc):
        for j in range(8):
            v = raw_v[pl.ds(r * _STREAM + j * 16, 16)]
            idx2d[r, pl.ds(j * 16, 16)] = jnp.maximum(v, 0)
            acc = jnp.minimum(acc, v)
        return acc
    min_acc = lax.fori_loop(0, _IDX_ROWS, _sanitize,
                            jnp.zeros((16,), jnp.int32))
    lane_min = min_acc[0]
    for _j in range(1, 16):
        lane_min = jnp.minimum(lane_min, min_acc[_j])

    ov = [oov_v[pl.ds(c * 16, 16)] for c in range(4)]

    def _fixup(g, buf):
        @pl.when(lane_min < 0)
        def _():
            def _chunk(k, carry):
                v = raw_v[pl.ds(g * _ROWS_PER_GROUP + k * 16, 16)]
                for j in range(16):
                    @pl.when(v[j] < 0)
                    def _():
                        for c in range(4):
                            buf[k * 16 + j, pl.ds(c * 16, 16)] = ov[c]
                return carry
            lax.fori_loop(0, _ROWS_PER_GROUP // 16, _chunk, jnp.int32(0))

    bufs = (rows0, rows1, rows2, rows3)
    gsems = (gsem0, gsem1, gsem2, gsem3)
    osems = (osem0, osem1, osem2, osem3)
    gathers = [None] * _NG
    outcopies = [None] * _NG

    def _drain_and_emit(g):
        buf = bufs[g % _NBUF]
        for h in gathers[g]:
            h.wait()
        _fixup(g, buf)
        outcopies[g] = pltpu.async_copy(
            buf, out_hbm.at[pl.ds(base + g * _ROWS_PER_GROUP, _ROWS_PER_GROUP)],
            osems[g % _NBUF])

    for g in range(_NG):
        b = g % _NBUF
        if g >= _NBUF:
            outcopies[g - _NBUF].wait()
        gathers[g] = [
            pltpu.async_copy(
                table_hbm.at[idx2d.at[g * _SPG + j]],
                bufs[b].at[pl.ds(j * _STREAM, _STREAM)],
                gsems[b])
            for j in range(_SPG)
        ]
        if g >= 1:
            _drain_and_emit(g - 1)
    _drain_and_emit(_NG - 1)
    for g in range(_NG - _NBUF, _NG):
        outcopies[g].wait()


def kernel(arr, table, oov):
    mesh = plsc.VectorSubcoreMesh(core_axis_name="c", subcore_axis_name="s")
    kern = functools.partial(
        pl.kernel,
        out_type=jax.ShapeDtypeStruct((_N, _DIM), jnp.float32),
        mesh=mesh,
        compiler_params=pltpu.CompilerParams(use_tc_tiling_on_sc=False),
        scratch_types=[
            pltpu.VMEM((_BPW,), jnp.int32),
            pltpu.VMEM((_IDX_ROWS, _STREAM), jnp.int32),
            pltpu.VMEM((_ROWS_PER_GROUP, _DIM), jnp.float32),
            pltpu.VMEM((_ROWS_PER_GROUP, _DIM), jnp.float32),
            pltpu.VMEM((_ROWS_PER_GROUP, _DIM), jnp.float32),
            pltpu.VMEM((_ROWS_PER_GROUP, _DIM), jnp.float32),
            pltpu.VMEM((_DIM,), jnp.float32),
            pltpu.SemaphoreType.DMA,
            pltpu.SemaphoreType.DMA,
            pltpu.SemaphoreType.DMA,
            pltpu.SemaphoreType.DMA,
            pltpu.SemaphoreType.DMA,
            pltpu.SemaphoreType.DMA,
            pltpu.SemaphoreType.DMA,
            pltpu.SemaphoreType.DMA,
        ],
    )(_body)
    out = kern(arr.reshape(-1), table, oov)
    return out.reshape(_BATCH, _HIST, _DIM)


# speedup vs baseline: 3.1816x; 1.0128x over previous
import functools

import jax
import jax.numpy as jnp
from jax import lax
from jax.experimental import pallas as pl
from jax.experimental.pallas import tpu as pltpu
from jax.experimental.pallas import tpu_sc as plsc

_VOCAB = 100000
_DIM = 64
_BATCH = 4096
_HIST = 50
_N = _BATCH * _HIST

_NC, _NS = 2, 16
_NW = _NC * _NS
_BPW = _N // _NW
_STREAM = 128
_ROWS_PER_GROUP = 640
_SPG = _ROWS_PER_GROUP // _STREAM
_NG = _BPW // _ROWS_PER_GROUP
_IDX_ROWS = _BPW // _STREAM


def _body(arr_hbm, table_hbm, oov_hbm, out_hbm,
          raw_v, idx2d, rows0, rows1, oov_v,
          gsem0, gsem1, osem0, osem1):
    wid = lax.axis_index("s") * _NC + lax.axis_index("c")
    base = wid * _BPW

    pltpu.sync_copy(arr_hbm.at[pl.ds(base, _BPW)], raw_v)
    pltpu.sync_copy(oov_hbm, oov_v)

    def _sanitize(r, acc):
        for j in range(8):
            v = raw_v[pl.ds(r * _STREAM + j * 16, 16)]
            idx2d[r, pl.ds(j * 16, 16)] = jnp.maximum(v, 0)
            acc = jnp.minimum(acc, v)
        return acc
    min_acc = lax.fori_loop(0, _IDX_ROWS, _sanitize,
                            jnp.zeros((16,), jnp.int32))
    lane_min = min_acc[0]
    for _j in range(1, 16):
        lane_min = jnp.minimum(lane_min, min_acc[_j])

    ov = [oov_v[pl.ds(c * 16, 16)] for c in range(4)]

    def _fixup(g, buf):
        @pl.when(lane_min < 0)
        def _():
            def _chunk(k, carry):
                v = raw_v[pl.ds(g * _ROWS_PER_GROUP + k * 16, 16)]
                for j in range(16):
                    @pl.when(v[j] < 0)
                    def _():
                        for c in range(4):
                            buf[k * 16 + j, pl.ds(c * 16, 16)] = ov[c]
                return carry
            lax.fori_loop(0, _ROWS_PER_GROUP // 16, _chunk, jnp.int32(0))

    bufs = (rows0, rows1)
    gsems = (gsem0, gsem1)
    osems = (osem0, osem1)
    gathers = [None] * _NG
    outcopies = [None] * _NG

    def _drain_and_emit(g):
        buf = bufs[g % 2]
        for h in gathers[g]:
            h.wait()
        _fixup(g, buf)
        outcopies[g] = pltpu.async_copy(
            buf, out_hbm.at[pl.ds(base + g * _ROWS_PER_GROUP, _ROWS_PER_GROUP)],
            osems[g % 2])

    for g in range(_NG):
        b = g % 2
        if g >= 2:
            outcopies[g - 2].wait()
        gathers[g] = [
            pltpu.async_copy(
                table_hbm.at[idx2d.at[g * _SPG + j]],
                bufs[b].at[pl.ds(j * _STREAM, _STREAM)],
                gsems[b])
            for j in range(_SPG)
        ]
        if g >= 1:
            _drain_and_emit(g - 1)
    _drain_and_emit(_NG - 1)
    outcopies[_NG - 2].wait()
    outcopies[_NG - 1].wait()


def kernel(arr, table, oov):
    mesh = plsc.VectorSubcoreMesh(core_axis_name="c", subcore_axis_name="s")
    kern = functools.partial(
        pl.kernel,
        out_type=jax.ShapeDtypeStruct((_N, _DIM), jnp.float32),
        mesh=mesh,
        compiler_params=pltpu.CompilerParams(use_tc_tiling_on_sc=False),
        scratch_types=[
            pltpu.VMEM((_BPW,), jnp.int32),
            pltpu.VMEM((_IDX_ROWS, _STREAM), jnp.int32),
            pltpu.VMEM((_ROWS_PER_GROUP, _DIM), jnp.float32),
            pltpu.VMEM((_ROWS_PER_GROUP, _DIM), jnp.float32),
            pltpu.VMEM((_DIM,), jnp.float32),
            pltpu.SemaphoreType.DMA,
            pltpu.SemaphoreType.DMA,
            pltpu.SemaphoreType.DMA,
            pltpu.SemaphoreType.DMA,
        ],
    )(_body)
    out = kern(arr.reshape(-1), table, oov)
    return out.reshape(_BATCH, _HIST, _DIM)


# speedup vs baseline: 4.4372x; 1.3947x over previous
import functools

import jax
import jax.numpy as jnp
from jax import lax
from jax.experimental import pallas as pl
from jax.experimental.pallas import tpu as pltpu
from jax.experimental.pallas import tpu_sc as plsc

_VOCAB = 100000
_DIM = 64
_BATCH = 4096
_HIST = 50
_N = _BATCH * _HIST

_NC, _NS = 2, 16
_NW = _NC * _NS
_BPW = _N // _NW
_STREAM = 128
_ROWS_PER_GROUP = 640
_SPG = _ROWS_PER_GROUP // _STREAM
_NG = _BPW // _ROWS_PER_GROUP
_IDX_ROWS = _BPW // _STREAM


def _body(arr_hbm, table_hbm, oov_hbm, out_hbm,
          raw_v, idx2d, rows0, rows1, oov_v,
          gsem0, gsem1, osem0, osem1):
    wid = lax.axis_index("s") * _NC + lax.axis_index("c")
    base = wid * _BPW

    pltpu.sync_copy(arr_hbm.at[pl.ds(base, _BPW)], raw_v)
    pltpu.sync_copy(oov_hbm, oov_v)

    def _sanitize(r, acc):
        for j in range(8):
            v = raw_v[pl.ds(r * _STREAM + j * 16, 16)]
            idx2d[r, pl.ds(j * 16, 16)] = jnp.maximum(v, 0)
            acc = jnp.minimum(acc, v)
        return acc
    min_acc = lax.fori_loop(0, _IDX_ROWS, _sanitize,
                            jnp.zeros((16,), jnp.int32))
    lane_min = min_acc[0]
    for _j in range(1, 16):
        lane_min = jnp.minimum(lane_min, min_acc[_j])

    ov = [oov_v[pl.ds(c * 16, 16)] for c in range(4)]

    def _fixup(g, buf):
        @pl.when(lane_min < 0)
        def _():
            def _chunk(k, carry):
                v = raw_v[pl.ds(g * _ROWS_PER_GROUP + k * 16, 16)]
                for j in range(16):
                    @pl.when(v[j] < 0)
                    def _():
                        for c in range(4):
                            buf[k * 16 + j, pl.ds(c * 16, 16)] = ov[c]
                return carry
            lax.fori_loop(0, _ROWS_PER_GROUP // 16, _chunk, jnp.int32(0))

    bufs = (rows0, rows1)
    gsems = (gsem0, gsem1)
    osems = (osem0, osem1)
    gathers = [None] * _NG
    outcopies = [None] * _NG

    def _drain_and_emit(g):
        buf = bufs[g % 2]
        for h in gathers[g]:
            h.wait()
        _fixup(g, buf)
        outcopies[g] = pltpu.async_copy(
            buf, out_hbm.at[pl.ds(base + g * _ROWS_PER_GROUP, _ROWS_PER_GROUP)],
            osems[g % 2])

    for g in range(_NG):
        b = g % 2
        if g >= 2:
            outcopies[g - 2].wait()
        gathers[g] = [
            pltpu.async_copy(
                table_hbm.at[idx2d.at[g * _SPG + j]],
                bufs[b].at[pl.ds(j * _STREAM, _STREAM)],
                gsems[b])
            for j in range(_SPG)
        ]
        if g >= 1:
            _drain_and_emit(g - 1)
    _drain_and_emit(_NG - 1)
    outcopies[_NG - 2].wait()
    outcopies[_NG - 1].wait()


def _relayout_body(x_ref, o_ref):
    x3 = x_ref[...].reshape(128, 3200 // _STREAM, _STREAM)
    t = jnp.transpose(x3, (1, 0, 2))
    t = jnp.transpose(t, (0, 2, 1))
    o_ref[...] = t.reshape(_HIST, 8, 1, 8, _STREAM)


def kernel(arr, table, oov):
    mesh = plsc.VectorSubcoreMesh(core_axis_name="c", subcore_axis_name="s")
    kern = functools.partial(
        pl.kernel,
        out_type=jax.ShapeDtypeStruct((_N, _DIM), jnp.float32),
        mesh=mesh,
        compiler_params=pltpu.CompilerParams(use_tc_tiling_on_sc=False),
        scratch_types=[
            pltpu.VMEM((_BPW,), jnp.int32),
            pltpu.VMEM((_IDX_ROWS, _STREAM), jnp.int32),
            pltpu.VMEM((_ROWS_PER_GROUP, _DIM), jnp.float32),
            pltpu.VMEM((_ROWS_PER_GROUP, _DIM), jnp.float32),
            pltpu.VMEM((_DIM,), jnp.float32),
            pltpu.SemaphoreType.DMA,
            pltpu.SemaphoreType.DMA,
            pltpu.SemaphoreType.DMA,
            pltpu.SemaphoreType.DMA,
        ],
    )(_body)
    lin = kern(arr.reshape(-1), table, oov)
    out5 = pl.pallas_call(
        _relayout_body,
        grid=(_NW,),
        in_specs=[pl.BlockSpec((3200, _STREAM), lambda w: (w, 0))],
        out_specs=pl.BlockSpec((_HIST, 8, 1, 8, _STREAM),
                               lambda w: (0, 0, w, 0, 0)),
        out_shape=jax.ShapeDtypeStruct((_HIST, 8, _NW, 8, _STREAM),
                                       jnp.float32),
    )(lin.reshape(_N * _DIM // _STREAM, _STREAM))
    return jnp.transpose(out5, (2, 4, 0, 1, 3)).reshape(_BATCH, _HIST, _DIM)


# speedup vs baseline: 4.7942x; 1.0805x over previous
import functools

import jax
import jax.numpy as jnp
from jax import lax
from jax.experimental import pallas as pl
from jax.experimental.pallas import tpu as pltpu
from jax.experimental.pallas import tpu_sc as plsc

_VOCAB = 100000
_DIM = 64
_BATCH = 4096
_HIST = 50
_N = _BATCH * _HIST

_NC, _NS = 2, 16
_NW = _NC * _NS
_BPW = _N // _NW
_STREAM = 128
_ROWS_PER_GROUP = 640
_SPG = _ROWS_PER_GROUP // _STREAM
_NG = _BPW // _ROWS_PER_GROUP
_IDX_ROWS = _BPW // _STREAM


def _body(arr_hbm, table_hbm, oov_hbm, out_hbm,
          raw_v, idx2d, rows0, rows1, oov_v,
          gsem0, gsem1, osem0, osem1):
    wid = lax.axis_index("s") * _NC + lax.axis_index("c")
    base = wid * _BPW

    pltpu.sync_copy(arr_hbm.at[pl.ds(base, _BPW)], raw_v)
    pltpu.sync_copy(oov_hbm, oov_v)

    def _sanitize(r, acc):
        for j in range(8):
            v = raw_v[pl.ds(r * _STREAM + j * 16, 16)]
            s = jnp.maximum(v, 0)
            g = ((lax.shift_right_logical(s, 11) * 2048)
                 + 2 * (s & 1023)
                 + (lax.shift_right_logical(s, 10) & 1))
            idx2d[r, pl.ds(j * 16, 16)] = g
            acc = jnp.minimum(acc, v)
        return acc
    min_acc = lax.fori_loop(0, _IDX_ROWS, _sanitize,
                            jnp.zeros((16,), jnp.int32))
    lane_min = min_acc[0]
    for _j in range(1, 16):
        lane_min = jnp.minimum(lane_min, min_acc[_j])

    ov = [oov_v[pl.ds(c * 16, 16)] for c in range(4)]

    def _fixup(g, buf):
        @pl.when(lane_min < 0)
        def _():
            def _chunk(k, carry):
                v = raw_v[pl.ds(g * _ROWS_PER_GROUP + k * 16, 16)]
                for j in range(16):
                    @pl.when(v[j] < 0)
                    def _():
                        for c in range(4):
                            buf[k * 16 + j, pl.ds(c * 16, 16)] = ov[c]
                return carry
            lax.fori_loop(0, _ROWS_PER_GROUP // 16, _chunk, jnp.int32(0))

    bufs = (rows0, rows1)
    gsems = (gsem0, gsem1)
    osems = (osem0, osem1)
    gathers = [None] * _NG
    outcopies = [None] * _NG

    def _drain_and_emit(g):
        buf = bufs[g % 2]
        for h in gathers[g]:
            h.wait()
        _fixup(g, buf)
        outcopies[g] = pltpu.async_copy(
            buf, out_hbm.at[pl.ds(base + g * _ROWS_PER_GROUP, _ROWS_PER_GROUP)],
            osems[g % 2])

    for g in range(_NG):
        b = g % 2
        if g >= 2:
            outcopies[g - 2].wait()
        gathers[g] = [
            pltpu.async_copy(
                table_hbm.at[idx2d.at[g * _SPG + j]],
                bufs[b].at[pl.ds(j * _STREAM, _STREAM)],
                gsems[b])
            for j in range(_SPG)
        ]
        if g >= 1:
            _drain_and_emit(g - 1)
    _drain_and_emit(_NG - 1)
    outcopies[_NG - 2].wait()
    outcopies[_NG - 1].wait()


def _tab_body(x_ref, o_ref):
    y = jnp.transpose(x_ref[...], (1, 0))
    o_ref[:, 0:64] = y[0:1024, :]
    o_ref[:, 64:128] = y[1024:2048, :]


def _relayout_body(x_ref, o_ref):
    x3 = x_ref[...].reshape(128, 3200 // _STREAM, _STREAM)
    t = jnp.transpose(x3, (1, 0, 2))
    t = jnp.transpose(t, (0, 2, 1))
    o_ref[...] = t.reshape(_HIST, 8, 1, 8, _STREAM)


def kernel(arr, table, oov):
    mesh = plsc.VectorSubcoreMesh(core_axis_name="c", subcore_axis_name="s")
    kern = functools.partial(
        pl.kernel,
        out_type=jax.ShapeDtypeStruct((_N, _DIM), jnp.float32),
        mesh=mesh,
        compiler_params=pltpu.CompilerParams(use_tc_tiling_on_sc=False),
        scratch_types=[
            pltpu.VMEM((_BPW,), jnp.int32),
            pltpu.VMEM((_IDX_ROWS, _STREAM), jnp.int32),
            pltpu.VMEM((_ROWS_PER_GROUP, _DIM), jnp.float32),
            pltpu.VMEM((_ROWS_PER_GROUP, _DIM), jnp.float32),
            pltpu.VMEM((_DIM,), jnp.float32),
            pltpu.SemaphoreType.DMA,
            pltpu.SemaphoreType.DMA,
            pltpu.SemaphoreType.DMA,
            pltpu.SemaphoreType.DMA,
        ],
    )(_body)
    table_lin = pl.pallas_call(
        _tab_body,
        grid=(49,),
        in_specs=[pl.BlockSpec((_DIM, 2048), lambda w: (0, w))],
        out_specs=pl.BlockSpec((1024, 128), lambda w: (w, 0)),
        out_shape=jax.ShapeDtypeStruct((49 * 1024, 128), jnp.float32),
    )(table.T)
    lin = kern(arr.reshape(-1), table_lin.reshape(49 * 2048, _DIM), oov)
    out5 = pl.pallas_call(
        _relayout_body,
        grid=(_NW,),
        in_specs=[pl.BlockSpec((3200, _STREAM), lambda w: (w, 0))],
        out_specs=pl.BlockSpec((_HIST, 8, 1, 8, _STREAM),
                               lambda w: (0, 0, w, 0, 0)),
        out_shape=jax.ShapeDtypeStruct((_HIST, 8, _NW, 8, _STREAM),
                                       jnp.float32),
    )(lin.reshape(_N * _DIM // _STREAM, _STREAM))
    return jnp.transpose(out5, (2, 4, 0, 1, 3)).reshape(_BATCH, _HIST, _DIM)
